# Initial kernel scaffold; baseline (speedup 1.0000x reference)
import jax
import jax.numpy as jnp
from jax.experimental import pallas as pl


def kernel(x_stroke, edge_index_temp_previous, edge_index_intersects, Wm_tp1, Wr_tp1, b_tp1, Wm_in1, Wr_in1, b_in1, Wm_tp2, Wr_tp2, b_tp2, Wm_in2, Wr_in2, b_in2, W_cls, b_cls):
    raise NotImplementedError("write your pallas kernel here")


# speedup vs baseline: 3.5903x; 3.5903x over previous
import jax
import jax.numpy as jnp
from jax import lax
from jax.experimental import pallas as pl
from jax.experimental.pallas import tpu as pltpu
from jax.experimental.pallas import tpu_sc as plsc

N = 100000
E = 800000
NS = 16
CH = 128
EP = E // NS
NF = EP // CH
REM = EP - NF * CH
STRIPE = N // NS
FS = 6256
FL = N - (NS - 1) * FS
ZR = 1250

ECP = E // 2 // NS
CNF = ECP // CH
CREM = ECP - CNF * CH

_SC_PARAMS = pltpu.CompilerParams(use_tc_tiling_on_sc=False)


def _zero_acc(z16, zbuf, acc, s):
  pltpu.sync_copy(z16, zbuf)
  for k in range(STRIPE // ZR):
    pltpu.sync_copy(zbuf, acc.at[pl.ds(s * STRIPE + k * ZR, ZR)])


def _flush_acc(acc, out, c, s):
  @pl.when(s < NS - 1)
  def _flush_main():
    fstripe = pl.ds(s * FS, FS)
    pltpu.sync_copy(acc.at[fstripe], out.at[c, fstripe])

  @pl.when(s == NS - 1)
  def _flush_tail():
    fstripe = pl.ds((NS - 1) * FS, FL)
    pltpu.sync_copy(acc.at[fstripe], out.at[c, fstripe])


def _make_scatter():
  mesh = plsc.VectorSubcoreMesh(core_axis_name="c", subcore_axis_name="s")
  out_type = jax.ShapeDtypeStruct((2, N, 16), jnp.float32)
  scratch = [
      pltpu.VMEM_SHARED((N, 16), jnp.float32),
      pltpu.VMEM((CH,), jnp.int32),
      pltpu.VMEM((CH,), jnp.int32),
      pltpu.VMEM((CH, 16), jnp.float32),
      pltpu.VMEM((REM,), jnp.int32),
      pltpu.VMEM((REM,), jnp.int32),
      pltpu.VMEM((REM, 16), jnp.float32),
      pltpu.VMEM((ZR, 16), jnp.float32),
  ]

  def body(table, gsrc, dst, z16, out, acc,
           dst_b, gidx, rows, dst_r, gidx_r, rows_r, zbuf):
    c = lax.axis_index("c")
    s = lax.axis_index("s")
    base = s * EP

    _zero_acc(z16, zbuf, acc, s)
    plsc.subcore_barrier()

    def chunk(off, nb, dst_c, gidx_c, rows_c):
      pltpu.sync_copy(gsrc.at[c, pl.ds(off, nb)], gidx_c)
      pltpu.sync_copy(dst.at[pl.ds(off, nb)], dst_c)
      pltpu.sync_copy(table.at[gidx_c], rows_c)
      pltpu.sync_copy(rows_c, acc.at[dst_c], add=True)

    def loop_body(g, carry):
      chunk(base + g * CH, CH, dst_b, gidx, rows)
      return carry

    lax.fori_loop(0, NF, loop_body, 0)
    chunk(base + NF * CH, REM, dst_r, gidx_r, rows_r)

    plsc.subcore_barrier()
    _flush_acc(acc, out, c, s)

  return pl.kernel(body, out_type=out_type, mesh=mesh, scratch_types=scratch,
                   compiler_params=_SC_PARAMS, name="sc_scatter")


_scatter = _make_scatter()


def _make_count():
  mesh = plsc.VectorSubcoreMesh(core_axis_name="c", subcore_axis_name="s")
  out_type = jax.ShapeDtypeStruct((2, N, 16), jnp.float32)
  scratch = [
      pltpu.VMEM_SHARED((N, 16), jnp.float32),
      pltpu.VMEM((CH,), jnp.int32),
      pltpu.VMEM((CH, 16), jnp.float32),
      pltpu.VMEM((CREM,), jnp.int32),
      pltpu.VMEM((CREM, 16), jnp.float32),
      pltpu.VMEM((ZR, 16), jnp.float32),
  ]

  def body(dst, z16, o16, out, cacc, dst_b, ones_b, dst_r, ones_r, zbuf):
    c = lax.axis_index("c")
    s = lax.axis_index("s")
    base = c * (E // 2) + s * ECP

    _zero_acc(z16, zbuf, cacc, s)
    pltpu.sync_copy(o16, ones_b)
    pltpu.sync_copy(o16.at[pl.ds(0, CREM)], ones_r)
    plsc.subcore_barrier()

    def chunk(off, dst_c, ones_c):
      pltpu.sync_copy(dst.at[pl.ds(off, dst_c.shape[0])], dst_c)
      pltpu.sync_copy(ones_c, cacc.at[dst_c], add=True)

    def loop_body(g, carry):
      chunk(base + g * CH, dst_b, ones_b)
      return carry

    lax.fori_loop(0, CNF, loop_body, 0)
    chunk(base + CNF * CH, dst_r, ones_r)

    plsc.subcore_barrier()
    _flush_acc(cacc, out, c, s)

  return pl.kernel(body, out_type=out_type, mesh=mesh, scratch_types=scratch,
                   compiler_params=_SC_PARAMS, name="sc_count")


_count = _make_count()

BM = 1024
_PREC = jax.lax.Precision.HIGHEST


def _dot(a, b):
  return jnp.dot(a, b, preferred_element_type=jnp.float32, precision=_PREC)


def _dense_common(x_ref, stp_ref, sin_ref, cnt_ref, wt_ref, wi_ref, wr_ref,
                  b_ref):
  x = x_ref[...]
  wt = wt_ref[...]
  wi = wi_ref[...]
  a = _dot(stp_ref[0], wt[:16]) + _dot(stp_ref[1], wt[16:])
  m = _dot(sin_ref[0], wi[:16]) + _dot(sin_ref[1], wi[16:])
  cnt = cnt_ref[0, :, 0] + cnt_ref[1, :, 0]
  rec = (1.0 / jnp.maximum(cnt, 1.0))[:, None]
  r = _dot(x, wr_ref[...])
  return jnp.maximum(a + m * rec + r + b_ref[...], 0.0) + x


def _dense_body(x_ref, stp_ref, sin_ref, cnt_ref, wt_ref, wi_ref, wr_ref,
                b_ref, o_ref):
  o_ref[...] = _dense_common(x_ref, stp_ref, sin_ref, cnt_ref, wt_ref,
                             wi_ref, wr_ref, b_ref)


def _dense_cls_body(x_ref, stp_ref, sin_ref, cnt_ref, wt_ref, wi_ref, wr_ref,
                    b_ref, wc_ref, bc_ref, o_ref):
  h = _dense_common(x_ref, stp_ref, sin_ref, cnt_ref, wt_ref, wi_ref, wr_ref,
                    b_ref)
  o_ref[...] = _dot(h, wc_ref[...]) + bc_ref[...]


def _dense_call(cls):
  grid = (pl.cdiv(N, BM),)
  in_specs = [
      pl.BlockSpec((BM, 32), lambda i: (i, 0)),
      pl.BlockSpec((2, BM, 16), lambda i: (0, i, 0)),
      pl.BlockSpec((2, BM, 16), lambda i: (0, i, 0)),
      pl.BlockSpec((2, BM, 16), lambda i: (0, i, 0)),
      pl.BlockSpec((32, 32), lambda i: (0, 0)),
      pl.BlockSpec((32, 32), lambda i: (0, 0)),
      pl.BlockSpec((32, 32), lambda i: (0, 0)),
      pl.BlockSpec((1, 32), lambda i: (0, 0)),
  ]
  if cls:
    in_specs += [
        pl.BlockSpec((32, 8), lambda i: (0, 0)),
        pl.BlockSpec((1, 8), lambda i: (0, 0)),
    ]
    out_spec = pl.BlockSpec((BM, 8), lambda i: (i, 0))
    out_shape = jax.ShapeDtypeStruct((N, 8), jnp.float32)
    body = _dense_cls_body
  else:
    out_spec = pl.BlockSpec((BM, 32), lambda i: (i, 0))
    out_shape = jax.ShapeDtypeStruct((N, 32), jnp.float32)
    body = _dense_body
  return pl.pallas_call(body, grid=grid, in_specs=in_specs,
                        out_specs=out_spec, out_shape=out_shape)


_dense1 = _dense_call(False)
_dense2 = _dense_call(True)


def kernel(x_stroke, edge_index_temp_previous, edge_index_intersects,
           Wm_tp1, Wr_tp1, b_tp1, Wm_in1, Wr_in1, b_in1,
           Wm_tp2, Wr_tp2, b_tp2, Wm_in2, Wr_in2, b_in2,
           W_cls, b_cls):
  src_tp = edge_index_temp_previous[0]
  dst_tp = edge_index_temp_previous[1]
  src_in = edge_index_intersects[0]
  dst_in = edge_index_intersects[1]
  gsrc_tp = jnp.stack([src_tp * 2, src_tp * 2 + 1])
  gsrc_in = jnp.stack([src_in * 2, src_in * 2 + 1])
  z16 = jnp.zeros((ZR, 16), jnp.float32)
  o16 = jnp.ones((CH, 16), jnp.float32)

  table1 = x_stroke.reshape(2 * N, 16)
  stp1 = _scatter(table1, gsrc_tp, dst_tp, z16)
  sin1 = _scatter(table1, gsrc_in, dst_in, z16)
  cnt3 = _count(dst_in, z16, o16)

  h1 = _dense1(x_stroke, stp1, sin1, cnt3, Wm_tp1, Wm_in1,
               Wr_tp1 + Wr_in1, (b_tp1 + b_in1).reshape(1, 32))

  table2 = h1.reshape(2 * N, 16)
  stp2 = _scatter(table2, gsrc_tp, dst_tp, z16)
  sin2 = _scatter(table2, gsrc_in, dst_in, z16)

  wc = jnp.zeros((32, 8), jnp.float32).at[:, :7].set(W_cls)
  bc = jnp.zeros((1, 8), jnp.float32).at[0, :7].set(b_cls)
  out = _dense2(h1, stp2, sin2, cnt3, Wm_tp2, Wm_in2,
                Wr_tp2 + Wr_in2, (b_tp2 + b_in2).reshape(1, 32), wc, bc)
  return out[:, :7]


# speedup vs baseline: 7.5731x; 2.1093x over previous
import jax
import jax.numpy as jnp
from jax import lax
from jax.experimental import pallas as pl
from jax.experimental.pallas import tpu as pltpu
from jax.experimental.pallas import tpu_sc as plsc

N = 100000
E = 800000
NS = 16
CH = 128
EP = E // NS
NF = EP // CH
REM = EP - NF * CH
STRIPE = N // NS
FS = 6256
FL = N - (NS - 1) * FS
ZR = 250

ECP = E // 2 // NS
CNF = ECP // CH
CREM = ECP - CNF * CH

_SC_PARAMS = pltpu.CompilerParams(use_tc_tiling_on_sc=False)


def _zero_acc(z16, zbuf, acc, s):
  pltpu.sync_copy(z16, zbuf)
  for k in range(STRIPE // ZR):
    pltpu.sync_copy(zbuf, acc.at[pl.ds(s * STRIPE + k * ZR, ZR)])


def _flush_acc(acc, out, c, s):
  @pl.when(s < NS - 1)
  def _flush_main():
    fstripe = pl.ds(s * FS, FS)
    pltpu.sync_copy(acc.at[fstripe], out.at[c, fstripe])

  @pl.when(s == NS - 1)
  def _flush_tail():
    fstripe = pl.ds((NS - 1) * FS, FL)
    pltpu.sync_copy(acc.at[fstripe], out.at[c, fstripe])


ROWS = E // CH
RPT = ROWS // NS
K = 8
NG = RPT // K
LEFT = RPT - NG * K
TAIL = ROWS - NS * RPT


def _make_scatter():
  mesh = plsc.VectorSubcoreMesh(core_axis_name="c", subcore_axis_name="s")
  out_type = jax.ShapeDtypeStruct((2, N, 16), jnp.float32)
  scratch = [
      pltpu.VMEM_SHARED((N, 16), jnp.float32),
      pltpu.VMEM((K, CH), jnp.int32),
      pltpu.VMEM((K, CH), jnp.int32),
      pltpu.VMEM((K, CH, 16), jnp.float32),
      pltpu.VMEM((ZR, 16), jnp.float32),
      pltpu.SemaphoreType.DMA,
      pltpu.SemaphoreType.DMA,
  ]

  def body(table, gsrc3, dst3h, z16, out, acc, gidx, dstb, rows,
           zbuf, semg, semsc):
    c = lax.axis_index("c")
    s = lax.axis_index("s")
    rowbase = s * RPT

    _zero_acc(z16, zbuf, acc, s)
    plsc.subcore_barrier()

    def loop_body(g, carry):
      roff = rowbase + g * K
      pltpu.sync_copy(gsrc3.at[c, pl.ds(roff, K)], gidx)
      pltpu.sync_copy(dst3h.at[pl.ds(roff, K)], dstb)
      gd = [pltpu.async_copy(table.at[gidx.at[j]], rows.at[j], semg)
            for j in range(K)]
      for d in gd:
        d.wait()
      sd = [pltpu.async_copy(rows.at[j], acc.at[dstb.at[j]], semsc, add=True)
            for j in range(K)]
      for d in sd:
        d.wait()
      return carry

    lax.fori_loop(0, NG, loop_body, 0)

    def sync_row(r):
      pltpu.sync_copy(gsrc3.at[c, r], gidx.at[0])
      pltpu.sync_copy(dst3h.at[r], dstb.at[0])
      pltpu.sync_copy(table.at[gidx.at[0]], rows.at[0])
      pltpu.sync_copy(rows.at[0], acc.at[dstb.at[0]], add=True)

    for t in range(LEFT):
      sync_row(rowbase + NG * K + t)

    @pl.when(s < TAIL)
    def _tail():
      sync_row(NS * RPT + s)

    plsc.subcore_barrier()
    _flush_acc(acc, out, c, s)

  return pl.kernel(body, out_type=out_type, mesh=mesh, scratch_types=scratch,
                   compiler_params=_SC_PARAMS, name="sc_scatter")


_scatter = _make_scatter()


def _make_count():
  mesh = plsc.VectorSubcoreMesh(core_axis_name="c", subcore_axis_name="s")
  out_type = jax.ShapeDtypeStruct((2, N, 16), jnp.float32)
  scratch = [
      pltpu.VMEM_SHARED((N, 16), jnp.float32),
      pltpu.VMEM((CH,), jnp.int32),
      pltpu.VMEM((CH, 16), jnp.float32),
      pltpu.VMEM((CREM,), jnp.int32),
      pltpu.VMEM((CREM, 16), jnp.float32),
      pltpu.VMEM((ZR, 16), jnp.float32),
  ]

  def body(dst, z16, o16, out, cacc, dst_b, ones_b, dst_r, ones_r, zbuf):
    c = lax.axis_index("c")
    s = lax.axis_index("s")
    base = c * (E // 2) + s * ECP

    _zero_acc(z16, zbuf, cacc, s)
    pltpu.sync_copy(o16, ones_b)
    pltpu.sync_copy(o16.at[pl.ds(0, CREM)], ones_r)
    plsc.subcore_barrier()

    def chunk(off, dst_c, ones_c):
      pltpu.sync_copy(dst.at[pl.ds(off, dst_c.shape[0])], dst_c)
      pltpu.sync_copy(ones_c, cacc.at[dst_c], add=True)

    def loop_body(g, carry):
      chunk(base + g * CH, dst_b, ones_b)
      return carry

    lax.fori_loop(0, CNF, loop_body, 0)
    chunk(base + CNF * CH, dst_r, ones_r)

    plsc.subcore_barrier()
    _flush_acc(cacc, out, c, s)

  return pl.kernel(body, out_type=out_type, mesh=mesh, scratch_types=scratch,
                   compiler_params=_SC_PARAMS, name="sc_count")


_count = _make_count()

BM = 1024
_PREC = jax.lax.Precision.HIGHEST


def _dot(a, b):
  return jnp.dot(a, b, preferred_element_type=jnp.float32, precision=_PREC)


def _dense_common(x_ref, stp_ref, sin_ref, cnt_ref, wt_ref, wi_ref, wr_ref,
                  b_ref):
  x = x_ref[...]
  wt = wt_ref[...]
  wi = wi_ref[...]
  a = _dot(stp_ref[0], wt[:16]) + _dot(stp_ref[1], wt[16:])
  m = _dot(sin_ref[0], wi[:16]) + _dot(sin_ref[1], wi[16:])
  cnt = cnt_ref[0, :, 0] + cnt_ref[1, :, 0]
  rec = (1.0 / jnp.maximum(cnt, 1.0))[:, None]
  r = _dot(x, wr_ref[...])
  return jnp.maximum(a + m * rec + r + b_ref[...], 0.0) + x


def _dense_body(x_ref, stp_ref, sin_ref, cnt_ref, wt_ref, wi_ref, wr_ref,
                b_ref, o_ref):
  o_ref[...] = _dense_common(x_ref, stp_ref, sin_ref, cnt_ref, wt_ref,
                             wi_ref, wr_ref, b_ref)


def _dense_cls_body(x_ref, stp_ref, sin_ref, cnt_ref, wt_ref, wi_ref, wr_ref,
                    b_ref, wc_ref, bc_ref, o_ref):
  h = _dense_common(x_ref, stp_ref, sin_ref, cnt_ref, wt_ref, wi_ref, wr_ref,
                    b_ref)
  o_ref[...] = _dot(h, wc_ref[...]) + bc_ref[...]


def _dense_call(cls):
  grid = (pl.cdiv(N, BM),)
  in_specs = [
      pl.BlockSpec((BM, 32), lambda i: (i, 0)),
      pl.BlockSpec((2, BM, 16), lambda i: (0, i, 0)),
      pl.BlockSpec((2, BM, 16), lambda i: (0, i, 0)),
      pl.BlockSpec((2, BM, 16), lambda i: (0, i, 0)),
      pl.BlockSpec((32, 32), lambda i: (0, 0)),
      pl.BlockSpec((32, 32), lambda i: (0, 0)),
      pl.BlockSpec((32, 32), lambda i: (0, 0)),
      pl.BlockSpec((1, 32), lambda i: (0, 0)),
  ]
  if cls:
    in_specs += [
        pl.BlockSpec((32, 8), lambda i: (0, 0)),
        pl.BlockSpec((1, 8), lambda i: (0, 0)),
    ]
    out_spec = pl.BlockSpec((BM, 8), lambda i: (i, 0))
    out_shape = jax.ShapeDtypeStruct((N, 8), jnp.float32)
    body = _dense_cls_body
  else:
    out_spec = pl.BlockSpec((BM, 32), lambda i: (i, 0))
    out_shape = jax.ShapeDtypeStruct((N, 32), jnp.float32)
    body = _dense_body
  return pl.pallas_call(body, grid=grid, in_specs=in_specs,
                        out_specs=out_spec, out_shape=out_shape)


_dense1 = _dense_call(False)
_dense2 = _dense_call(True)


def kernel(x_stroke, edge_index_temp_previous, edge_index_intersects,
           Wm_tp1, Wr_tp1, b_tp1, Wm_in1, Wr_in1, b_in1,
           Wm_tp2, Wr_tp2, b_tp2, Wm_in2, Wr_in2, b_in2,
           W_cls, b_cls):
  src_tp = edge_index_temp_previous[0]
  dst_tp = edge_index_temp_previous[1]
  src_in = edge_index_intersects[0]
  dst_in = edge_index_intersects[1]
  gsrc_tp = jnp.stack([src_tp * 2, src_tp * 2 + 1]).reshape(2, ROWS, CH)
  gsrc_in = jnp.stack([src_in * 2, src_in * 2 + 1]).reshape(2, ROWS, CH)
  dst_tp = dst_tp.reshape(ROWS, CH)
  dst_in3 = dst_in.reshape(ROWS, CH)
  z16 = jnp.zeros((ZR, 16), jnp.float32)
  o16 = jnp.ones((CH, 16), jnp.float32)

  table1 = x_stroke.reshape(2 * N, 16)
  stp1 = _scatter(table1, gsrc_tp, dst_tp, z16)
  sin1 = _scatter(table1, gsrc_in, dst_in3, z16)
  cnt3 = _count(dst_in, z16, o16)

  h1 = _dense1(x_stroke, stp1, sin1, cnt3, Wm_tp1, Wm_in1,
               Wr_tp1 + Wr_in1, (b_tp1 + b_in1).reshape(1, 32))

  table2 = h1.reshape(2 * N, 16)
  stp2 = _scatter(table2, gsrc_tp, dst_tp, z16)
  sin2 = _scatter(table2, gsrc_in, dst_in3, z16)

  wc = jnp.zeros((32, 8), jnp.float32).at[:, :7].set(W_cls)
  bc = jnp.zeros((1, 8), jnp.float32).at[0, :7].set(b_cls)
  out = _dense2(h1, stp2, sin2, cnt3, Wm_tp2, Wm_in2,
                Wr_tp2 + Wr_in2, (b_tp2 + b_in2).reshape(1, 32), wc, bc)
  return out[:, :7]


# speedup vs baseline: 10.9872x; 1.4508x over previous
import jax
import jax.numpy as jnp
from jax import lax
from jax.experimental import pallas as pl
from jax.experimental.pallas import tpu as pltpu
from jax.experimental.pallas import tpu_sc as plsc

N = 100000
E = 800000
NS = 16
CH = 128
EP = E // NS
NF = EP // CH
REM = EP - NF * CH
STRIPE = N // NS
FS = 6256
FL = N - (NS - 1) * FS
ZR = 250

ECP = E // 2 // NS
CNF = ECP // CH
CREM = ECP - CNF * CH

_SC_PARAMS = pltpu.CompilerParams(use_tc_tiling_on_sc=False)


def _zero_acc(z16, zbuf, acc, s):
  pltpu.sync_copy(z16, zbuf)
  for k in range(STRIPE // ZR):
    pltpu.sync_copy(zbuf, acc.at[pl.ds(s * STRIPE + k * ZR, ZR)])


def _flush_acc(acc, out, c, s):
  @pl.when(s < NS - 1)
  def _flush_main():
    fstripe = pl.ds(s * FS, FS)
    pltpu.sync_copy(acc.at[fstripe], out.at[fstripe, c])

  @pl.when(s == NS - 1)
  def _flush_tail():
    fstripe = pl.ds((NS - 1) * FS, FL)
    pltpu.sync_copy(acc.at[fstripe], out.at[fstripe, c])


ROWS = E // CH
RPT = ROWS // NS
K = 8
NG = RPT // K
LEFT = RPT - NG * K
TAIL = ROWS - NS * RPT


def _make_layer(do_count):
  mesh = plsc.VectorSubcoreMesh(core_axis_name="c", subcore_axis_name="s")
  out_type = [jax.ShapeDtypeStruct((N, 2, 16), jnp.float32),
              jax.ShapeDtypeStruct((N, 2, 16), jnp.float32)]
  if do_count:
    out_type.append(jax.ShapeDtypeStruct((N, 2, 16), jnp.float32))
  scratch = [
      pltpu.VMEM_SHARED((N, 16), jnp.float32),
      pltpu.VMEM((K, CH), jnp.int32),
      pltpu.VMEM((K, CH), jnp.int32),
      pltpu.VMEM((K, CH), jnp.int32),
      pltpu.VMEM((K, CH, 16), jnp.float32),
      pltpu.VMEM((ZR, 16), jnp.float32),
      pltpu.VMEM((CH, 16), jnp.float32),
      pltpu.SemaphoreType.DMA,
      pltpu.SemaphoreType.DMA,
      pltpu.SemaphoreType.DMA,
  ]

  def body(*refs):
    if do_count:
      (table, src_tp3, dst_tp3, src_in3, dst_in3, z16, o16,
       out_tp, out_in, out_cnt,
       acc, srcb, gidx, dstb, rows, zbuf, onesb, semg, semsc, semz) = refs
    else:
      (table, src_tp3, dst_tp3, src_in3, dst_in3, z16, o16,
       out_tp, out_in,
       acc, srcb, gidx, dstb, rows, zbuf, onesb, semg, semsc, semz) = refs

    c = lax.axis_index("c")
    s = lax.axis_index("s")

    pltpu.sync_copy(o16, onesb)

    def zero_acc():
      pltpu.sync_copy(z16, zbuf)
      zd = [pltpu.async_copy(zbuf, acc.at[pl.ds(s * STRIPE + k * ZR, ZR)],
                             semz) for k in range(STRIPE // ZR)]
      for d in zd:
        d.wait()

    def compute_gidx(nrows):
      for j in range(nrows):
        for v in range(CH // 16):
          sl = pl.ds(v * 16, 16)
          gidx[j, sl] = srcb[j, sl] * 2 + c

    def scatter_group(src3, dst3, roff, nrows):
      pltpu.sync_copy(src3.at[pl.ds(roff, nrows)], srcb.at[pl.ds(0, nrows)])
      pltpu.sync_copy(dst3.at[pl.ds(roff, nrows)], dstb.at[pl.ds(0, nrows)])
      compute_gidx(nrows)
      gd = [pltpu.async_copy(table.at[gidx.at[j]], rows.at[j], semg)
            for j in range(nrows)]
      for d in gd:
        d.wait()
      sd = [pltpu.async_copy(rows.at[j], acc.at[dstb.at[j]], semsc, add=True)
            for j in range(nrows)]
      for d in sd:
        d.wait()

    def scatter_phase(src3, dst3, out):
      zero_acc()
      plsc.subcore_barrier()
      rowbase = s * RPT

      def loop_body(g, carry):
        scatter_group(src3, dst3, rowbase + g * K, K)
        return carry

      lax.fori_loop(0, NG, loop_body, 0)
      scatter_group(src3, dst3, rowbase + NG * K, LEFT)

      @pl.when(s < TAIL)
      def _tail():
        scatter_group(src3, dst3, NS * RPT + s, 1)

      plsc.subcore_barrier()
      _flush_acc(acc, out, c, s)
      plsc.subcore_barrier()

    def count_group(dst3, roff, nrows):
      pltpu.sync_copy(dst3.at[pl.ds(roff, nrows)], dstb.at[pl.ds(0, nrows)])
      sd = [pltpu.async_copy(onesb, acc.at[dstb.at[j]], semsc, add=True)
            for j in range(nrows)]
      for d in sd:
        d.wait()

    def count_phase(dst3, out):
      zero_acc()
      plsc.subcore_barrier()
      rowbase = s * RPT

      def loop_body(g, carry):
        count_group(dst3, rowbase + g * K, K)
        return carry

      lax.fori_loop(0, NG, loop_body, 0)
      count_group(dst3, rowbase + NG * K, LEFT)

      @pl.when(s < TAIL)
      def _tail():
        count_group(dst3, NS * RPT + s, 1)

      plsc.subcore_barrier()
      _flush_acc(acc, out, c, s)

    scatter_phase(src_tp3, dst_tp3, out_tp)
    scatter_phase(src_in3, dst_in3, out_in)
    if do_count:
      count_phase(dst_in3, out_cnt)

  return pl.kernel(body, out_type=out_type, mesh=mesh, scratch_types=scratch,
                   compiler_params=_SC_PARAMS,
                   name="sc_layer_cnt" if do_count else "sc_layer")


_layer1 = _make_layer(True)
_layer2 = _make_layer(False)

N4 = N // 4
BM4 = 2048


def _dense_common(x_ref, stp_ref, sin_ref, cnt_ref, wt_ref, wi_ref, wr_ref,
                  b_ref):
  x = x_ref[...]
  a = jnp.dot(stp_ref[...], wt_ref[...], preferred_element_type=jnp.float32,
              precision=jax.lax.Precision.HIGHEST)
  m = jnp.dot(sin_ref[...], wi_ref[...], preferred_element_type=jnp.float32,
              precision=jax.lax.Precision.HIGHEST)
  rec = 1.0 / jnp.maximum(cnt_ref[...], 1.0)
  r = jnp.dot(x, wr_ref[...], preferred_element_type=jnp.float32,
              precision=jax.lax.Precision.HIGHEST)
  return jnp.maximum(a + m * rec + r + b_ref[...], 0.0) + x


def _dense_body(x_ref, stp_ref, sin_ref, cnt_ref, wt_ref, wi_ref, wr_ref,
                b_ref, o_ref):
  o_ref[...] = _dense_common(x_ref, stp_ref, sin_ref, cnt_ref, wt_ref,
                             wi_ref, wr_ref, b_ref)


def _dense_cls_body(x_ref, stp_ref, sin_ref, cnt_ref, wt_ref, wi_ref, wr_ref,
                    b_ref, wc_ref, bc_ref, o_ref):
  h = _dense_common(x_ref, stp_ref, sin_ref, cnt_ref, wt_ref, wi_ref, wr_ref,
                    b_ref)
  o_ref[...] = jnp.dot(h, wc_ref[...],
                       preferred_element_type=jnp.float32,
              precision=jax.lax.Precision.HIGHEST) + bc_ref[...]


def _dense_call(cls):
  grid = (pl.cdiv(N4, BM4),)
  in_specs = [
      pl.BlockSpec((BM4, 128), lambda i: (i, 0)),
      pl.BlockSpec((BM4, 128), lambda i: (i, 0)),
      pl.BlockSpec((BM4, 128), lambda i: (i, 0)),
      pl.BlockSpec((BM4, 128), lambda i: (i, 0)),
      pl.BlockSpec((128, 128), lambda i: (0, 0)),
      pl.BlockSpec((128, 128), lambda i: (0, 0)),
      pl.BlockSpec((128, 128), lambda i: (0, 0)),
      pl.BlockSpec((1, 128), lambda i: (0, 0)),
  ]
  if cls:
    in_specs += [
        pl.BlockSpec((128, 32), lambda i: (0, 0)),
        pl.BlockSpec((1, 32), lambda i: (0, 0)),
    ]
    out_spec = pl.BlockSpec((BM4, 32), lambda i: (i, 0))
    out_shape = jax.ShapeDtypeStruct((N4, 32), jnp.float32)
    body = _dense_cls_body
  else:
    out_spec = pl.BlockSpec((BM4, 128), lambda i: (i, 0))
    out_shape = jax.ShapeDtypeStruct((N4, 128), jnp.float32)
    body = _dense_body
  return pl.pallas_call(body, grid=grid, in_specs=in_specs,
                        out_specs=out_spec, out_shape=out_shape)


_dense1 = _dense_call(False)
_dense2 = _dense_call(True)


def _kron4(w):
  return jnp.kron(jnp.eye(4, dtype=jnp.float32), w)


def kernel(x_stroke, edge_index_temp_previous, edge_index_intersects,
           Wm_tp1, Wr_tp1, b_tp1, Wm_in1, Wr_in1, b_in1,
           Wm_tp2, Wr_tp2, b_tp2, Wm_in2, Wr_in2, b_in2,
           W_cls, b_cls):
  src_tp3 = edge_index_temp_previous[0].reshape(ROWS, CH)
  dst_tp3 = edge_index_temp_previous[1].reshape(ROWS, CH)
  src_in3 = edge_index_intersects[0].reshape(ROWS, CH)
  dst_in3 = edge_index_intersects[1].reshape(ROWS, CH)
  z16 = jnp.zeros((ZR, 16), jnp.float32)
  o16 = jnp.ones((CH, 16), jnp.float32)

  table1 = x_stroke.reshape(2 * N, 16)
  stp1, sin1, cnt3 = _layer1(table1, src_tp3, dst_tp3, src_in3, dst_in3,
                             z16, o16)
  cnt4 = cnt3.reshape(N4, 128)

  x4 = x_stroke.reshape(N4, 128)
  h1 = _dense1(x4, stp1.reshape(N4, 128), sin1.reshape(N4, 128), cnt4,
               _kron4(Wm_tp1), _kron4(Wm_in1), _kron4(Wr_tp1 + Wr_in1),
               jnp.tile(b_tp1 + b_in1, 4).reshape(1, 128))

  table2 = h1.reshape(2 * N, 16)
  stp2, sin2 = _layer2(table2, src_tp3, dst_tp3, src_in3, dst_in3, z16, o16)

  wc = jnp.zeros((32, 8), jnp.float32).at[:, :7].set(W_cls)
  bc = jnp.zeros((8,), jnp.float32).at[:7].set(b_cls)
  out4 = _dense2(h1, stp2.reshape(N4, 128), sin2.reshape(N4, 128), cnt4,
                 _kron4(Wm_tp2), _kron4(Wm_in2), _kron4(Wr_tp2 + Wr_in2),
                 jnp.tile(b_tp2 + b_in2, 4).reshape(1, 128),
                 _kron4(wc), jnp.tile(bc, 4).reshape(1, 32))
  return out4.reshape(N, 8)[:, :7]
